# baseline (speedup 1.0000x reference)
import dataclasses
import functools

import jax
import jax.numpy as jnp
from jax import lax
from jax.experimental import pallas as pl
from jax.experimental.pallas import tpu as pltpu
from jax.experimental.pallas import tpu_sc as plsc

BM = 1024
BN = 1024

def _sc_mesh():
    return plsc.VectorSubcoreMesh(core_axis_name="c", subcore_axis_name="s")

_SC_PARAMS = pltpu.CompilerParams()
if "needs_layout_passes" in pltpu.CompilerParams.__dataclass_fields__:
    _SC_PARAMS = dataclasses.replace(_SC_PARAMS, needs_layout_passes=False)


def _sim_body(hid_ref, pat_ref, sim_ref, top_ref, hn_ref, pn_ref, lmax_ref, lvc_ref):
    i = pl.program_id(0)
    j = pl.program_id(1)
    nj = pl.num_programs(1)

    @pl.when(j == 0)
    def _():
        h = hid_ref[...]
        nrm = jnp.sqrt(jnp.sum(h * h, axis=1, keepdims=True))
        hn_ref[...] = h / jnp.maximum(nrm, 1e-8)
        lmax_ref[...] = jnp.full((BM, 128), -jnp.inf, jnp.float32)
        lvc_ref[...] = jnp.zeros((BM, 128), jnp.int32)

    @pl.when(i == 0)
    def _():
        p = pat_ref[...]
        nrm = jnp.sqrt(jnp.sum(p * p, axis=0, keepdims=True))
        pn_ref[:, pl.ds(j * BN, BN)] = p / jnp.maximum(nrm, 1e-8)

    sim = jnp.dot(hn_ref[...], pn_ref[:, pl.ds(j * BN, BN)],
                  preferred_element_type=jnp.float32)
    sim_ref[...] = sim
    lmax = lmax_ref[...]
    lvc = lvc_ref[...]
    for vc in range(BN // 128):
        s = sim[:, vc * 128:(vc + 1) * 128]
        better = s > lmax
        lmax = jnp.where(better, s, lmax)
        lvc = jnp.where(better, jnp.full((BM, 128), j * (BN // 128) + vc,
                                         jnp.int32), lvc)
    lmax_ref[...] = lmax
    lvc_ref[...] = lvc

    @pl.when(j == nj - 1)
    def _():
        gidx = lvc * 128 + lax.broadcasted_iota(jnp.int32, (BM, 128), 1)
        rowmax = jnp.max(lmax, axis=1, keepdims=True)
        top_ref[...] = jnp.min(jnp.where(lmax == rowmax, gidx, BM * BN), axis=1)


def _sim_argmax(hidden_states, pattern_vectors):
    B, D = hidden_states.shape
    K = pattern_vectors.shape[0]
    pat_t = pattern_vectors.T
    grid = (B // BM, K // BN)
    return pl.pallas_call(
        _sim_body,
        grid=grid,
        in_specs=[
            pl.BlockSpec((BM, D), lambda i, j: (i, 0)),
            pl.BlockSpec((D, BN), lambda i, j: (0, j)),
        ],
        out_specs=[
            pl.BlockSpec((BM, BN), lambda i, j: (i, j)),
            pl.BlockSpec((BM,), lambda i, j: (i,)),
        ],
        out_shape=[
            jax.ShapeDtypeStruct((B, K), jnp.float32),
            jax.ShapeDtypeStruct((B,), jnp.int32),
        ],
        scratch_shapes=[
            pltpu.VMEM((BM, D), jnp.float32),
            pltpu.VMEM((D, K), jnp.float32),
            pltpu.VMEM((BM, 128), jnp.float32),
            pltpu.VMEM((BM, 128), jnp.int32),
        ],
    )(hidden_states, pat_t)


_NPAIR = 4096 * 19
_CCH = 8
_CPP = _NPAIR // _CCH
_CBINS = 2048


def _cooc_sc(p1, p2):
    @functools.partial(
        pl.kernel,
        mesh=_sc_mesh(),
        compiler_params=_SC_PARAMS,
        out_type=jax.ShapeDtypeStruct((65536,), jnp.float32),
        scratch_types=[
            pltpu.VMEM((_CPP,), jnp.int32),
            pltpu.VMEM((_CPP,), jnp.int32),
            pltpu.VMEM((_CBINS,), jnp.float32),
        ],
    )
    def k(p1_hbm, p2_hbm, out_hbm, pa_v, pb_v, tab_v):
        wid = lax.axis_index("c") * 16 + lax.axis_index("s")
        base = wid * _CBINS
        ones = jnp.full((16,), 1.0, jnp.float32)

        @pl.loop(0, _CBINS // 16)
        def _(q):
            tab_v[pl.ds(q * 16, 16)] = jnp.zeros((16,), jnp.float32)

        for ch in range(_CCH):
            pltpu.sync_copy(p1_hbm.at[pl.ds(ch * _CPP, _CPP)], pa_v)
            pltpu.sync_copy(p2_hbm.at[pl.ds(ch * _CPP, _CPP)], pb_v)

            @pl.loop(0, _CPP // 16)
            def _(q):
                a = pa_v[pl.ds(q * 16, 16)]
                b = pb_v[pl.ds(q * 16, 16)]
                g = a * 256 + b - base
                mask = (g >= 0) & (g < _CBINS)
                gc = jnp.where(mask, g, 0)
                plsc.addupdate_scatter(tab_v, [gc], ones, mask=mask)

        pltpu.sync_copy(tab_v, out_hbm.at[pl.ds(base, _CBINS)])

    return k(p1, p2)


_KPT = 256
_GCH = 128


def _usage_ctx_sc(top, hidden):
    B = top.shape[0]
    D = hidden.shape[1]

    @functools.partial(
        pl.kernel,
        mesh=_sc_mesh(),
        compiler_params=_SC_PARAMS,
        out_type=[
            jax.ShapeDtypeStruct((8192,), jnp.float32),
            jax.ShapeDtypeStruct((8192, D), jnp.float32),
        ],
        scratch_types=[
            pltpu.VMEM((B,), jnp.int32),
            pltpu.VMEM((B + 16,), jnp.int32),
            pltpu.VMEM((B + 16,), jnp.int32),
            pltpu.VMEM((_GCH, D), jnp.float32),
            pltpu.VMEM((_KPT,), jnp.float32),
            pltpu.VMEM((_KPT + 1, D), jnp.float32),
        ],
    )
    def k(top_hbm, hid_hbm, ouse_hbm, octx_hbm,
          idx_v, mypos_v, myloc_v, rows_v, use_v, ctx_v):
        wid = lax.axis_index("c") * 16 + lax.axis_index("s")
        kbase = wid * _KPT
        ones = jnp.full((16,), 1.0, jnp.float32)
        lane = lax.iota(jnp.int32, 16)

        with jax.named_scope("u_top_dma"):
            pltpu.sync_copy(top_hbm, idx_v)

        @pl.loop(0, _KPT // 16)
        def _(q):
            use_v[pl.ds(q * 16, 16)] = jnp.zeros((16,), jnp.float32)

        with jax.named_scope("u_zero"):
            @pl.loop(0, _KPT + 1)
            def _(r):
                for u in range(D // 16):
                    ctx_v[r, pl.ds(u * 16, 16)] = jnp.zeros((16,), jnp.float32)

        @pl.loop(0, (B + 16) // 16)
        def _(q):
            mypos_v[pl.ds(q * 16, 16)] = jnp.zeros((16,), jnp.int32)
            myloc_v[pl.ds(q * 16, 16)] = jnp.full((16,), _KPT, jnp.int32)

        with jax.named_scope("u_scan"):
            @pl.loop(0, B // 16, init_carry=0)
            def n(q, cnt):
                g = idx_v[pl.ds(q * 16, 16)] - kbase
                mask = (g >= 0) & (g < _KPT)
                gc = jnp.where(mask, g, 0)
                plsc.addupdate_scatter(use_v, [gc], ones, mask=mask)
                plsc.store_compressed(mypos_v.at[pl.ds(cnt, 16)],
                                      lane + q * 16, mask=mask)
                plsc.store_compressed(myloc_v.at[pl.ds(cnt, 16)],
                                      jnp.where(mask, g, _KPT), mask=mask)
                return cnt + jnp.sum(mask.astype(jnp.int32))

        nch = lax.div(n + (_GCH - 1), _GCH)

        with jax.named_scope("u_gather"):
            @pl.loop(0, nch)
            def _(w):
                with jax.named_scope("u_gdma"):
                    pltpu.sync_copy(
                        hid_hbm.at[mypos_v.at[pl.ds(w * _GCH, _GCH)]], rows_v)

                @pl.loop(0, _GCH // 16)
                def _(h):
                    locv = myloc_v[pl.ds(w * _GCH + h * 16, 16)]
                    for l in range(16):
                        r = locv[l]
                        for u in range(D // 16):
                            sl = pl.ds(u * 16, 16)
                            ctx_v[r, sl] = ctx_v[r, sl] + rows_v[h * 16 + l, sl]

        with jax.named_scope("u_out"):
            pltpu.sync_copy(use_v, ouse_hbm.at[pl.ds(kbase, _KPT)])
            pltpu.sync_copy(ctx_v.at[pl.ds(0, _KPT)],
                            octx_hbm.at[pl.ds(kbase, _KPT)])

    return k(top, hidden)


def kernel(phoneme_seq, hidden_states, pattern_vectors):
    similarity, top = _sim_argmax(hidden_states, pattern_vectors)

    p1 = phoneme_seq[:, :-1].reshape(-1)
    p2 = phoneme_seq[:, 1:].reshape(-1)
    cooccurrence = _cooc_sc(p1, p2).reshape(256, 256)

    usage_count, context_sum = _usage_ctx_sc(top, hidden_states)
    return (similarity, cooccurrence, usage_count, context_sum)


# speedup vs baseline: 1.2316x; 1.2316x over previous
import dataclasses
import functools

import jax
import jax.numpy as jnp
from jax import lax
from jax.experimental import pallas as pl
from jax.experimental.pallas import tpu as pltpu
from jax.experimental.pallas import tpu_sc as plsc

BM = 1024
BN = 1024

def _sc_mesh():
    return plsc.VectorSubcoreMesh(core_axis_name="c", subcore_axis_name="s")

_SC_PARAMS = pltpu.CompilerParams()
if "needs_layout_passes" in pltpu.CompilerParams.__dataclass_fields__:
    _SC_PARAMS = dataclasses.replace(_SC_PARAMS, needs_layout_passes=False)


def _sim_body(hid_ref, pat_ref, sim_ref, top_ref, hn_ref, pn_ref, lmax_ref, lvc_ref):
    i = pl.program_id(0)
    j = pl.program_id(1)
    nj = pl.num_programs(1)

    @pl.when(j == 0)
    def _():
        h = hid_ref[...]
        nrm = jnp.sqrt(jnp.sum(h * h, axis=1, keepdims=True))
        hn_ref[...] = h / jnp.maximum(nrm, 1e-8)
        lmax_ref[...] = jnp.full((BM, 128), -jnp.inf, jnp.float32)
        lvc_ref[...] = jnp.zeros((BM, 128), jnp.int32)

    @pl.when(i == 0)
    def _():
        p = pat_ref[...]
        nrm = jnp.sqrt(jnp.sum(p * p, axis=0, keepdims=True))
        pn_ref[:, pl.ds(j * BN, BN)] = p / jnp.maximum(nrm, 1e-8)

    sim = jnp.dot(hn_ref[...], pn_ref[:, pl.ds(j * BN, BN)],
                  preferred_element_type=jnp.float32)
    sim_ref[...] = sim
    lmax = lmax_ref[...]
    lvc = lvc_ref[...]
    for vc in range(BN // 128):
        s = sim[:, vc * 128:(vc + 1) * 128]
        better = s > lmax
        lmax = jnp.where(better, s, lmax)
        lvc = jnp.where(better, jnp.full((BM, 128), j * (BN // 128) + vc,
                                         jnp.int32), lvc)
    lmax_ref[...] = lmax
    lvc_ref[...] = lvc

    @pl.when(j == nj - 1)
    def _():
        gidx = lvc * 128 + lax.broadcasted_iota(jnp.int32, (BM, 128), 1)
        rowmax = jnp.max(lmax, axis=1, keepdims=True)
        top_ref[...] = jnp.min(jnp.where(lmax == rowmax, gidx, BM * BN), axis=1)


def _sim_argmax(hidden_states, pattern_vectors):
    B, D = hidden_states.shape
    K = pattern_vectors.shape[0]
    pat_t = pattern_vectors.T
    grid = (B // BM, K // BN)
    return pl.pallas_call(
        _sim_body,
        grid=grid,
        in_specs=[
            pl.BlockSpec((BM, D), lambda i, j: (i, 0)),
            pl.BlockSpec((D, BN), lambda i, j: (0, j)),
        ],
        out_specs=[
            pl.BlockSpec((BM, BN), lambda i, j: (i, j)),
            pl.BlockSpec((BM,), lambda i, j: (i,)),
        ],
        out_shape=[
            jax.ShapeDtypeStruct((B, K), jnp.float32),
            jax.ShapeDtypeStruct((B,), jnp.int32),
        ],
        scratch_shapes=[
            pltpu.VMEM((BM, D), jnp.float32),
            pltpu.VMEM((D, K), jnp.float32),
            pltpu.VMEM((BM, 128), jnp.float32),
            pltpu.VMEM((BM, 128), jnp.int32),
        ],
    )(hidden_states, pat_t)


_NPAIR = 4096 * 19
_CCH = 8
_CPP = _NPAIR // _CCH
_CBINS = 2048


def _cooc_sc(p1, p2):
    @functools.partial(
        pl.kernel,
        mesh=_sc_mesh(),
        compiler_params=_SC_PARAMS,
        out_type=jax.ShapeDtypeStruct((65536,), jnp.float32),
        scratch_types=[
            pltpu.VMEM((_CPP,), jnp.int32),
            pltpu.VMEM((_CPP,), jnp.int32),
            pltpu.VMEM((_CBINS,), jnp.float32),
        ],
    )
    def k(p1_hbm, p2_hbm, out_hbm, pa_v, pb_v, tab_v):
        wid = lax.axis_index("c") * 16 + lax.axis_index("s")
        base = wid * _CBINS
        ones = jnp.full((16,), 1.0, jnp.float32)

        @pl.loop(0, _CBINS // 16)
        def _(q):
            tab_v[pl.ds(q * 16, 16)] = jnp.zeros((16,), jnp.float32)

        for ch in range(_CCH):
            pltpu.sync_copy(p1_hbm.at[pl.ds(ch * _CPP, _CPP)], pa_v)
            pltpu.sync_copy(p2_hbm.at[pl.ds(ch * _CPP, _CPP)], pb_v)

            @pl.loop(0, _CPP // 16)
            def _(q):
                a = pa_v[pl.ds(q * 16, 16)]
                b = pb_v[pl.ds(q * 16, 16)]
                g = a * 256 + b - base
                mask = (g >= 0) & (g < _CBINS)
                gc = jnp.where(mask, g, 0)
                plsc.addupdate_scatter(tab_v, [gc], ones, mask=mask)

        pltpu.sync_copy(tab_v, out_hbm.at[pl.ds(base, _CBINS)])

    return k(p1, p2)


_KPT = 256
_GCH = 64


def _usage_ctx_sc(top, hidden):
    B = top.shape[0]
    D = hidden.shape[1]

    @functools.partial(
        pl.kernel,
        mesh=_sc_mesh(),
        compiler_params=_SC_PARAMS,
        out_type=[
            jax.ShapeDtypeStruct((8192,), jnp.float32),
            jax.ShapeDtypeStruct((8192, D), jnp.float32),
        ],
        scratch_types=[
            pltpu.VMEM((B,), jnp.int32),
            pltpu.VMEM((B + 16,), jnp.int32),
            pltpu.VMEM((B + 16,), jnp.int32),
            pltpu.VMEM((_GCH, D), jnp.float32),
            pltpu.VMEM((_GCH, D), jnp.float32),
            pltpu.VMEM((_KPT,), jnp.float32),
            pltpu.VMEM((_KPT + 1, D), jnp.float32),
            pltpu.SemaphoreType.DMA,
            pltpu.SemaphoreType.DMA,
        ],
    )
    def k(top_hbm, hid_hbm, ouse_hbm, octx_hbm,
          idx_v, mypos_v, myloc_v, rows_a, rows_b, use_v, ctx_v, sem_a, sem_b):
        wid = lax.axis_index("c") * 16 + lax.axis_index("s")
        kbase = wid * _KPT
        ones = jnp.full((16,), 1.0, jnp.float32)
        lane = lax.iota(jnp.int32, 16)

        with jax.named_scope("u_top_dma"):
            pltpu.sync_copy(top_hbm, idx_v)

        @pl.loop(0, _KPT // 16)
        def _(q):
            use_v[pl.ds(q * 16, 16)] = jnp.zeros((16,), jnp.float32)

        with jax.named_scope("u_zero"):
            @pl.loop(0, _KPT + 1)
            def _(r):
                for u in range(D // 16):
                    ctx_v[r, pl.ds(u * 16, 16)] = jnp.zeros((16,), jnp.float32)

        @pl.loop(0, (B + 16) // 16)
        def _(q):
            mypos_v[pl.ds(q * 16, 16)] = jnp.zeros((16,), jnp.int32)
            myloc_v[pl.ds(q * 16, 16)] = jnp.full((16,), _KPT, jnp.int32)

        with jax.named_scope("u_scan"):
            @pl.loop(0, B // 16, init_carry=0)
            def n(q, cnt):
                g = idx_v[pl.ds(q * 16, 16)] - kbase
                mask = (g >= 0) & (g < _KPT)
                gc = jnp.where(mask, g, 0)
                plsc.addupdate_scatter(use_v, [gc], ones, mask=mask)
                plsc.store_compressed(mypos_v.at[pl.ds(cnt, 16)],
                                      lane + q * 16, mask=mask)
                plsc.store_compressed(myloc_v.at[pl.ds(cnt, 16)],
                                      jnp.where(mask, g, _KPT), mask=mask)
                return cnt + jnp.sum(mask.astype(jnp.int32))

        nch = lax.div(n + (_GCH - 1), _GCH)

        def start(w, buf, sem):
            pltpu.async_copy(hid_hbm.at[mypos_v.at[pl.ds(w * _GCH, _GCH)]],
                             buf, sem)

        def wait(buf, sem):
            pltpu.make_async_copy(
                hid_hbm.at[mypos_v.at[pl.ds(0, _GCH)]], buf, sem).wait()

        def accum(w, buf):
            @pl.loop(0, _GCH // 16)
            def _(h):
                locv = myloc_v[pl.ds(w * _GCH + h * 16, 16)]
                for l in range(16):
                    r = locv[l]
                    for u in range(D // 16):
                        sl = pl.ds(u * 16, 16)
                        ctx_v[r, sl] = ctx_v[r, sl] + buf[h * 16 + l, sl]

        with jax.named_scope("u_gather"):
            @pl.when(nch > 0)
            def _():
                start(0, rows_a, sem_a)

            @pl.loop(0, lax.div(nch + 1, 2))
            def _(p):
                w0 = 2 * p
                wait(rows_a, sem_a)

                @pl.when(w0 + 1 < nch)
                def _():
                    start(w0 + 1, rows_b, sem_b)
                accum(w0, rows_a)

                @pl.when(w0 + 2 < nch)
                def _():
                    start(w0 + 2, rows_a, sem_a)

                @pl.when(w0 + 1 < nch)
                def _():
                    wait(rows_b, sem_b)
                    accum(w0 + 1, rows_b)

        with jax.named_scope("u_out"):
            pltpu.sync_copy(use_v, ouse_hbm.at[pl.ds(kbase, _KPT)])
            pltpu.sync_copy(ctx_v.at[pl.ds(0, _KPT)],
                            octx_hbm.at[pl.ds(kbase, _KPT)])

    return k(top, hidden)


def kernel(phoneme_seq, hidden_states, pattern_vectors):
    similarity, top = _sim_argmax(hidden_states, pattern_vectors)

    p1 = phoneme_seq[:, :-1].reshape(-1)
    p2 = phoneme_seq[:, 1:].reshape(-1)
    cooccurrence = _cooc_sc(p1, p2).reshape(256, 256)

    usage_count, context_sum = _usage_ctx_sc(top, hidden_states)
    return (similarity, cooccurrence, usage_count, context_sum)


# speedup vs baseline: 1.2460x; 1.0117x over previous
import dataclasses
import functools

import jax
import jax.numpy as jnp
from jax import lax
from jax.experimental import pallas as pl
from jax.experimental.pallas import tpu as pltpu
from jax.experimental.pallas import tpu_sc as plsc

BM = 1024
BN = 1024

def _sc_mesh():
    return plsc.VectorSubcoreMesh(core_axis_name="c", subcore_axis_name="s")

_SC_PARAMS = pltpu.CompilerParams()
if "needs_layout_passes" in pltpu.CompilerParams.__dataclass_fields__:
    _SC_PARAMS = dataclasses.replace(_SC_PARAMS, needs_layout_passes=False)


def _sim_body(hid_ref, pat_ref, sim_ref, top_ref, hn_ref, pn_ref, lmax_ref, lvc_ref):
    i = pl.program_id(0)
    j = pl.program_id(1)
    nj = pl.num_programs(1)

    @pl.when(j == 0)
    def _():
        h = hid_ref[...]
        nrm = jnp.sqrt(jnp.sum(h * h, axis=1, keepdims=True))
        hn_ref[...] = h / jnp.maximum(nrm, 1e-8)
        lmax_ref[...] = jnp.full((BM, 128), -jnp.inf, jnp.float32)
        lvc_ref[...] = jnp.zeros((BM, 128), jnp.int32)

    @pl.when(i == 0)
    def _():
        p = pat_ref[...]
        nrm = jnp.sqrt(jnp.sum(p * p, axis=0, keepdims=True))
        pn_ref[:, pl.ds(j * BN, BN)] = p / jnp.maximum(nrm, 1e-8)

    sim = jnp.dot(hn_ref[...], pn_ref[:, pl.ds(j * BN, BN)],
                  preferred_element_type=jnp.float32)
    sim_ref[...] = sim
    lmax = lmax_ref[...]
    lvc = lvc_ref[...]
    for vc in range(BN // 128):
        s = sim[:, vc * 128:(vc + 1) * 128]
        better = s > lmax
        lmax = jnp.where(better, s, lmax)
        lvc = jnp.where(better, jnp.full((BM, 128), j * (BN // 128) + vc,
                                         jnp.int32), lvc)
    lmax_ref[...] = lmax
    lvc_ref[...] = lvc

    @pl.when(j == nj - 1)
    def _():
        gidx = lvc * 128 + lax.broadcasted_iota(jnp.int32, (BM, 128), 1)
        rowmax = jnp.max(lmax, axis=1, keepdims=True)
        top_ref[...] = jnp.min(jnp.where(lmax == rowmax, gidx, BM * BN), axis=1)


def _sim_argmax(hidden_states, pattern_vectors):
    B, D = hidden_states.shape
    K = pattern_vectors.shape[0]
    pat_t = pattern_vectors.T
    grid = (B // BM, K // BN)
    return pl.pallas_call(
        _sim_body,
        grid=grid,
        in_specs=[
            pl.BlockSpec((BM, D), lambda i, j: (i, 0)),
            pl.BlockSpec((D, BN), lambda i, j: (0, j)),
        ],
        out_specs=[
            pl.BlockSpec((BM, BN), lambda i, j: (i, j)),
            pl.BlockSpec((BM,), lambda i, j: (i,)),
        ],
        out_shape=[
            jax.ShapeDtypeStruct((B, K), jnp.float32),
            jax.ShapeDtypeStruct((B,), jnp.int32),
        ],
        scratch_shapes=[
            pltpu.VMEM((BM, D), jnp.float32),
            pltpu.VMEM((D, K), jnp.float32),
            pltpu.VMEM((BM, 128), jnp.float32),
            pltpu.VMEM((BM, 128), jnp.int32),
        ],
    )(hidden_states, pat_t)


_NPAIR = 4096 * 19
_CCH = 8
_CPP = _NPAIR // _CCH
_CBINS = 2048


def _cooc_sc(p1, p2):
    @functools.partial(
        pl.kernel,
        mesh=_sc_mesh(),
        compiler_params=_SC_PARAMS,
        out_type=jax.ShapeDtypeStruct((65536,), jnp.float32),
        scratch_types=[
            pltpu.VMEM((_CPP,), jnp.int32),
            pltpu.VMEM((_CPP,), jnp.int32),
            pltpu.VMEM((_CBINS,), jnp.float32),
        ],
    )
    def k(p1_hbm, p2_hbm, out_hbm, pa_v, pb_v, tab_v):
        wid = lax.axis_index("c") * 16 + lax.axis_index("s")
        base = wid * _CBINS
        ones = jnp.full((16,), 1.0, jnp.float32)

        @pl.loop(0, _CBINS // 16)
        def _(q):
            tab_v[pl.ds(q * 16, 16)] = jnp.zeros((16,), jnp.float32)

        for ch in range(_CCH):
            pltpu.sync_copy(p1_hbm.at[pl.ds(ch * _CPP, _CPP)], pa_v)
            pltpu.sync_copy(p2_hbm.at[pl.ds(ch * _CPP, _CPP)], pb_v)

            @pl.loop(0, _CPP // 16)
            def _(q):
                a = pa_v[pl.ds(q * 16, 16)]
                b = pb_v[pl.ds(q * 16, 16)]
                g = a * 256 + b - base
                mask = (g >= 0) & (g < _CBINS)
                gc = jnp.where(mask, g, 0)
                plsc.addupdate_scatter(tab_v, [gc], ones, mask=mask)

        pltpu.sync_copy(tab_v, out_hbm.at[pl.ds(base, _CBINS)])

    return k(p1, p2)


_KPT = 256
_GCH = 64


def _usage_ctx_sc(top, hidden):
    B = top.shape[0]
    D = hidden.shape[1]

    @functools.partial(
        pl.kernel,
        mesh=_sc_mesh(),
        compiler_params=_SC_PARAMS,
        out_type=[
            jax.ShapeDtypeStruct((8192,), jnp.float32),
            jax.ShapeDtypeStruct((8192, D), jnp.float32),
        ],
        scratch_types=[
            pltpu.VMEM((B,), jnp.int32),
            pltpu.VMEM((B + 16,), jnp.int32),
            pltpu.VMEM((B + 16,), jnp.int32),
            pltpu.VMEM((_GCH, D), jnp.float32),
            pltpu.VMEM((_GCH, D), jnp.float32),
            pltpu.VMEM((_KPT,), jnp.float32),
            pltpu.VMEM((_KPT + 1, D), jnp.float32),
            pltpu.SemaphoreType.DMA,
            pltpu.SemaphoreType.DMA,
        ],
    )
    def k(top_hbm, hid_hbm, ouse_hbm, octx_hbm,
          idx_v, mypos_v, myloc_v, rows_a, rows_b, use_v, ctx_v, sem_a, sem_b):
        wid = lax.axis_index("c") * 16 + lax.axis_index("s")
        kbase = wid * _KPT
        ones = jnp.full((16,), 1.0, jnp.float32)
        lane = lax.iota(jnp.int32, 16)

        with jax.named_scope("u_top_dma"):
            pltpu.sync_copy(top_hbm, idx_v)

        @pl.loop(0, _KPT // 16)
        def _(q):
            use_v[pl.ds(q * 16, 16)] = jnp.zeros((16,), jnp.float32)

        with jax.named_scope("u_zero"):
            @pl.loop(0, _KPT + 1)
            def _(r):
                for u in range(D // 16):
                    ctx_v[r, pl.ds(u * 16, 16)] = jnp.zeros((16,), jnp.float32)

        @pl.loop(0, (B + 16) // 16)
        def _(q):
            mypos_v[pl.ds(q * 16, 16)] = jnp.zeros((16,), jnp.int32)
            myloc_v[pl.ds(q * 16, 16)] = jnp.full((16,), _KPT, jnp.int32)

        with jax.named_scope("u_scan"):
            @pl.loop(0, B // 16, init_carry=0)
            def n(q, cnt):
                g = idx_v[pl.ds(q * 16, 16)] - kbase
                mask = (g >= 0) & (g < _KPT)
                gc = jnp.where(mask, g, 0)
                plsc.addupdate_scatter(use_v, [gc], ones, mask=mask)
                plsc.store_compressed(mypos_v.at[pl.ds(cnt, 16)],
                                      lane + q * 16, mask=mask)
                plsc.store_compressed(myloc_v.at[pl.ds(cnt, 16)],
                                      jnp.where(mask, g, _KPT), mask=mask)
                return cnt + jnp.sum(mask.astype(jnp.int32))

        nch = lax.div(n + (_GCH - 1), _GCH)

        def start(w, buf, sem):
            pltpu.async_copy(hid_hbm.at[mypos_v.at[pl.ds(w * _GCH, _GCH)]],
                             buf, sem)

        def wait(buf, sem):
            pltpu.make_async_copy(
                hid_hbm.at[mypos_v.at[pl.ds(0, _GCH)]], buf, sem).wait()

        def accum(w, buf):
            @pl.loop(0, _GCH // 16)
            def _(h):
                locv = myloc_v[pl.ds(w * _GCH + h * 16, 16)]
                for l in range(16):
                    r = locv[l]

                    @pl.loop(0, D // 16)
                    def _(u):
                        sl = pl.ds(u * 16, 16)
                        ctx_v[r, sl] = ctx_v[r, sl] + buf[h * 16 + l, sl]

        with jax.named_scope("u_gather"):
            @pl.when(nch > 0)
            def _():
                start(0, rows_a, sem_a)

            @pl.loop(0, lax.div(nch + 1, 2))
            def _(p):
                w0 = 2 * p
                wait(rows_a, sem_a)

                @pl.when(w0 + 1 < nch)
                def _():
                    start(w0 + 1, rows_b, sem_b)
                accum(w0, rows_a)

                @pl.when(w0 + 2 < nch)
                def _():
                    start(w0 + 2, rows_a, sem_a)

                @pl.when(w0 + 1 < nch)
                def _():
                    wait(rows_b, sem_b)
                    accum(w0 + 1, rows_b)

        with jax.named_scope("u_out"):
            pltpu.sync_copy(use_v, ouse_hbm.at[pl.ds(kbase, _KPT)])
            pltpu.sync_copy(ctx_v.at[pl.ds(0, _KPT)],
                            octx_hbm.at[pl.ds(kbase, _KPT)])

    return k(top, hidden)


def kernel(phoneme_seq, hidden_states, pattern_vectors):
    similarity, top = _sim_argmax(hidden_states, pattern_vectors)

    p1 = phoneme_seq[:, :-1].reshape(-1)
    p2 = phoneme_seq[:, 1:].reshape(-1)
    cooccurrence = _cooc_sc(p1, p2).reshape(256, 256)

    usage_count, context_sum = _usage_ctx_sc(top, hidden_states)
    return (similarity, cooccurrence, usage_count, context_sum)
